# Initial kernel scaffold; baseline (speedup 1.0000x reference)
import jax
import jax.numpy as jnp
from jax.experimental import pallas as pl


def kernel(feat_3d, coords, training, W1, b1, W2, b2):
    raise NotImplementedError("write your pallas kernel here")


# speedup vs baseline: 1.0086x; 1.0086x over previous
import functools

import jax
import jax.numpy as jnp
from jax import lax
from jax.experimental import pallas as pl
from jax.experimental.pallas import tpu as pltpu
from jax.experimental.pallas import tpu_sc as plsc

_NW = 32
_GRID = 16
_NBINS = _GRID ** 3


def _minmax_body(coords_ref, params_ref):
    x = coords_ref[...]
    mn = jnp.min(x, axis=2)
    mx = jnp.max(x, axis=2)
    rg = mx - mn + 1e-6
    pad = jnp.zeros((x.shape[0], 1), jnp.float32)
    rows8 = jnp.concatenate([mn, pad, rg, pad], axis=1)
    params_ref[...] = jnp.broadcast_to(rows8[:, :, None], params_ref.shape)


def _featsum_body(feat_ref, acc_ref):
    @pl.when(pl.program_id(0) == 0)
    def _init():
        acc_ref[...] = jnp.zeros_like(acc_ref)

    acc_ref[...] += jnp.sum(feat_ref[...], axis=1)


def _softmax(x):
    m = jnp.max(x, axis=-1, keepdims=True)
    e = jnp.exp(x - m)
    return e / jnp.sum(e, axis=-1, keepdims=True)


def _final_body(hp_ref, sums_ref, w1_ref, b1_ref, w2_ref, b2_ref, gum_ref,
                lognf_ref, logits_ref, rwe_ref, rwt_ref, *, B, N, nparts):
    counts = hp_ref[...].reshape(B, nparts, _NBINS).sum(axis=1)
    hist = counts / (jnp.float32(N) + 1e-6)
    dmean = jnp.mean(hist, axis=1, keepdims=True)
    d = hist - dmean
    var = jnp.sum(d * d, axis=1, keepdims=True) / (_NBINS - 1)
    skew = jnp.mean(d * d * d, axis=1, keepdims=True) / (var * jnp.sqrt(var) + 1e-6)
    g_feat = sums_ref[...] / jnp.float32(N)
    ri = jnp.concatenate([g_feat, lognf_ref[...], dmean, var, skew], axis=1)
    dn = (((1,), (1,)), ((), ()))
    h = jnp.maximum(
        lax.dot_general(ri, w1_ref[...], dn, preferred_element_type=jnp.float32)
        + b1_ref[...], 0.0)
    logits = (lax.dot_general(h, w2_ref[...], dn, preferred_element_type=jnp.float32)
              + b2_ref[...])
    logits_ref[...] = logits
    rwe_ref[...] = _softmax(logits)
    rwt_ref[...] = _softmax(logits + gum_ref[...])


def _sc_hist_body(coords_hbm, params_hbm, out_hbm, cbuf, pbuf, hist,
                  *, npts, nper, N):
    nc = 2
    wid = lax.axis_index("s") * nc + lax.axis_index("c")
    b = wid // nper
    k = wid % nper
    for d in range(3):
        pltpu.sync_copy(
            coords_hbm.at[pl.ds((b * 3 + d) * N + k * npts, npts)],
            cbuf.at[pl.ds(d * npts, npts)])
    pltpu.sync_copy(params_hbm.at[pl.ds(b * 128, 128)], pbuf)

    zeros16 = jnp.zeros((16,), jnp.float32)

    def zero_body(j, carry):
        hist[pl.ds(j * 16, 16)] = zeros16
        return carry

    lax.fori_loop(0, _NBINS // 16, zero_body, 0)

    mnx = pbuf[pl.ds(0, 16)]
    mny = pbuf[pl.ds(16, 16)]
    mnz = pbuf[pl.ds(32, 16)]
    rgx = pbuf[pl.ds(64, 16)]
    rgy = pbuf[pl.ds(80, 16)]
    rgz = pbuf[pl.ds(96, 16)]
    ones16 = jnp.ones((16,), jnp.float32)
    g1 = jnp.float32(_GRID - 1)

    def body(i, carry):
        x = cbuf[pl.ds(i * 16, 16)]
        y = cbuf[pl.ds(npts + i * 16, 16)]
        z = cbuf[pl.ds(2 * npts + i * 16, 16)]
        ix = jnp.clip(((x - mnx) / rgx * g1).astype(jnp.int32), 0, _GRID - 1)
        iy = jnp.clip(((y - mny) / rgy * g1).astype(jnp.int32), 0, _GRID - 1)
        iz = jnp.clip(((z - mnz) / rgz * g1).astype(jnp.int32), 0, _GRID - 1)
        idx = ix * (_GRID * _GRID) + iy * _GRID + iz
        plsc.addupdate_scatter(hist, [idx], ones16)
        return carry

    lax.fori_loop(0, npts // 16, body, 0)
    pltpu.sync_copy(hist, out_hbm.at[pl.ds(wid * _NBINS, _NBINS)])


def _sc_hist(coords_flat, params_flat, B, N):
    npts = (B * N) // _NW
    nper = _NW // B
    mesh = plsc.VectorSubcoreMesh(core_axis_name="c", subcore_axis_name="s")
    body = functools.partial(_sc_hist_body, npts=npts, nper=nper, N=N)
    fn = pl.kernel(
        body,
        mesh=mesh,
        compiler_params=pltpu.CompilerParams(needs_layout_passes=False),
        out_type=jax.ShapeDtypeStruct((_NW * _NBINS,), jnp.float32),
        scratch_types=[
            pltpu.VMEM((3 * npts,), jnp.float32),
            pltpu.VMEM((128,), jnp.float32),
            pltpu.VMEM((_NBINS,), jnp.float32),
        ],
    )
    return fn(coords_flat, params_flat)


def kernel(feat_3d, coords, training, W1, b1, W2, b2):
    B, N, C = feat_3d.shape
    coords_t = jnp.transpose(coords, (0, 2, 1))

    params = pl.pallas_call(
        _minmax_body,
        out_shape=jax.ShapeDtypeStruct((B, 8, 16), jnp.float32),
    )(coords_t)

    hp = _sc_hist(coords_t.reshape(-1), params.reshape(-1), B, N)
    hp = hp.reshape(_NW, _NBINS)

    chunk = 1024
    sums = pl.pallas_call(
        _featsum_body,
        grid=(N // chunk,),
        in_specs=[pl.BlockSpec((B, chunk, C), lambda i: (0, i, 0))],
        out_specs=pl.BlockSpec((B, C), lambda i: (0, 0)),
        out_shape=jax.ShapeDtypeStruct((B, C), jnp.float32),
    )(feat_3d)

    log_n = (jnp.log(jnp.asarray(N, jnp.float32)) - 8.0) / 4.0
    lognf = jnp.broadcast_to(log_n.reshape(1, 1), (B, 1))
    u = jax.random.uniform(jax.random.key(42), (B, 3), dtype=jnp.float32)
    gumbel = -jnp.log(-jnp.log(u + 1e-10) + 1e-10)

    out_sd = jax.ShapeDtypeStruct((B, 3), jnp.float32)
    final = functools.partial(_final_body, B=B, N=N, nparts=_NW // B)
    logits, rwe, rwt = pl.pallas_call(
        final,
        out_shape=[out_sd, out_sd, out_sd],
    )(hp, sums, W1, b1.reshape(1, -1), W2, b2.reshape(1, -1), gumbel, lognf)

    routing_weights = jnp.where(training, rwt, rwe)
    return routing_weights, logits


# speedup vs baseline: 1.0348x; 1.0259x over previous
import functools

import jax
import jax.numpy as jnp
from jax import lax
from jax.experimental import pallas as pl
from jax.experimental.pallas import tpu as pltpu
from jax.experimental.pallas import tpu_sc as plsc

_NW = 32
_GRID = 16
_NBINS = _GRID ** 3


def _minmax_body(coords_ref, params_ref):
    x = coords_ref[...]
    mn = jnp.min(x, axis=2)
    mx = jnp.max(x, axis=2)
    rg = mx - mn + 1e-6
    pad = jnp.zeros((x.shape[0], 1), jnp.float32)
    rows8 = jnp.concatenate([mn, pad, rg, pad], axis=1)
    params_ref[...] = jnp.broadcast_to(rows8[:, :, None], params_ref.shape)


def _featsum_body(feat_ref, acc_ref):
    @pl.when(pl.program_id(0) == 0)
    def _init():
        acc_ref[...] = jnp.zeros_like(acc_ref)

    acc_ref[...] += jnp.sum(feat_ref[...], axis=1)


def _softmax(x):
    m = jnp.max(x, axis=-1, keepdims=True)
    e = jnp.exp(x - m)
    return e / jnp.sum(e, axis=-1, keepdims=True)


def _final_body(hp_ref, sums_ref, w1_ref, b1_ref, w2_ref, b2_ref, gum_ref,
                lognf_ref, logits_ref, rwe_ref, rwt_ref, *, B, N, nparts):
    counts = hp_ref[...].reshape(B, nparts, _NBINS).sum(axis=1)
    hist = counts / (jnp.float32(N) + 1e-6)
    dmean = jnp.mean(hist, axis=1, keepdims=True)
    d = hist - dmean
    var = jnp.sum(d * d, axis=1, keepdims=True) / (_NBINS - 1)
    skew = jnp.mean(d * d * d, axis=1, keepdims=True) / (var * jnp.sqrt(var) + 1e-6)
    g_feat = sums_ref[...] / jnp.float32(N)
    ri = jnp.concatenate([g_feat, lognf_ref[...], dmean, var, skew], axis=1)
    dn = (((1,), (1,)), ((), ()))
    h = jnp.maximum(
        lax.dot_general(ri, w1_ref[...], dn, preferred_element_type=jnp.float32)
        + b1_ref[...], 0.0)
    logits = (lax.dot_general(h, w2_ref[...], dn, preferred_element_type=jnp.float32)
              + b2_ref[...])
    logits_ref[...] = logits
    rwe_ref[...] = _softmax(logits)
    rwt_ref[...] = _softmax(logits + gum_ref[...])


def _sc_hist_body(coords_hbm, params_hbm, out_hbm, cbuf, pbuf, hist,
                  *, npts, nper, N):
    nc = 2
    wid = lax.axis_index("s") * nc + lax.axis_index("c")
    b = wid // nper
    k = wid % nper
    for d in range(3):
        pltpu.sync_copy(
            coords_hbm.at[pl.ds((b * 3 + d) * N + k * npts, npts)],
            cbuf.at[pl.ds(d * npts, npts)])
    pltpu.sync_copy(params_hbm.at[pl.ds(b * 128, 128)], pbuf)

    zeros16 = jnp.zeros((16,), jnp.float32)

    def zero_body(j, carry):
        hist[pl.ds(j * 16, 16)] = zeros16
        return carry

    lax.fori_loop(0, _NBINS // 16, zero_body, 0)

    mnx = pbuf[pl.ds(0, 16)]
    mny = pbuf[pl.ds(16, 16)]
    mnz = pbuf[pl.ds(32, 16)]
    rgx = pbuf[pl.ds(64, 16)]
    rgy = pbuf[pl.ds(80, 16)]
    rgz = pbuf[pl.ds(96, 16)]
    ones16 = jnp.ones((16,), jnp.float32)
    g1 = jnp.float32(_GRID - 1)

    def body(i, carry):
        x = cbuf[pl.ds(i * 16, 16)]
        y = cbuf[pl.ds(npts + i * 16, 16)]
        z = cbuf[pl.ds(2 * npts + i * 16, 16)]
        ix = jnp.clip(((x - mnx) / rgx * g1).astype(jnp.int32), 0, _GRID - 1)
        iy = jnp.clip(((y - mny) / rgy * g1).astype(jnp.int32), 0, _GRID - 1)
        iz = jnp.clip(((z - mnz) / rgz * g1).astype(jnp.int32), 0, _GRID - 1)
        idx = ix * (_GRID * _GRID) + iy * _GRID + iz
        plsc.addupdate_scatter(hist, [idx], ones16)
        return carry

    lax.fori_loop(0, npts // 16, body, 0)
    pltpu.sync_copy(hist, out_hbm.at[pl.ds(wid * _NBINS, _NBINS)])


def _sc_hist(coords_flat, params_flat, B, N):
    npts = (B * N) // _NW
    nper = _NW // B
    mesh = plsc.VectorSubcoreMesh(core_axis_name="c", subcore_axis_name="s")
    body = functools.partial(_sc_hist_body, npts=npts, nper=nper, N=N)
    fn = pl.kernel(
        body,
        mesh=mesh,
        compiler_params=pltpu.CompilerParams(needs_layout_passes=False),
        out_type=jax.ShapeDtypeStruct((_NW * _NBINS,), jnp.float32),
        scratch_types=[
            pltpu.VMEM((3 * npts,), jnp.float32),
            pltpu.VMEM((128,), jnp.float32),
            pltpu.VMEM((_NBINS,), jnp.float32),
        ],
    )
    return fn(coords_flat, params_flat)


def kernel(feat_3d, coords, training, W1, b1, W2, b2):
    B, N, C = feat_3d.shape
    coords_t = jnp.transpose(coords, (0, 2, 1))

    params = pl.pallas_call(
        _minmax_body,
        out_shape=jax.ShapeDtypeStruct((B, 8, 16), jnp.float32),
    )(coords_t)

    hp = _sc_hist(coords_t.reshape(-1), params.reshape(-1), B, N)
    hp = hp.reshape(_NW, _NBINS)

    chunk = 4096
    sums = pl.pallas_call(
        _featsum_body,
        grid=(N // chunk,),
        in_specs=[pl.BlockSpec((B, chunk, C), lambda i: (0, i, 0))],
        out_specs=pl.BlockSpec((B, C), lambda i: (0, 0)),
        out_shape=jax.ShapeDtypeStruct((B, C), jnp.float32),
    )(feat_3d)

    log_n = (jnp.log(jnp.asarray(N, jnp.float32)) - 8.0) / 4.0
    lognf = jnp.broadcast_to(log_n.reshape(1, 1), (B, 1))
    u = jax.random.uniform(jax.random.key(42), (B, 3), dtype=jnp.float32)
    gumbel = -jnp.log(-jnp.log(u + 1e-10) + 1e-10)

    out_sd = jax.ShapeDtypeStruct((B, 3), jnp.float32)
    final = functools.partial(_final_body, B=B, N=N, nparts=_NW // B)
    logits, rwe, rwt = pl.pallas_call(
        final,
        out_shape=[out_sd, out_sd, out_sd],
    )(hp, sums, W1, b1.reshape(1, -1), W2, b2.reshape(1, -1), gumbel, lognf)

    routing_weights = jnp.where(training, rwt, rwe)
    return routing_weights, logits


# speedup vs baseline: 1.0382x; 1.0034x over previous
import functools

import jax
import jax.numpy as jnp
from jax import lax
from jax.experimental import pallas as pl
from jax.experimental.pallas import tpu as pltpu
from jax.experimental.pallas import tpu_sc as plsc

_NW = 32
_GRID = 16
_NBINS = _GRID ** 3


def _minmax_body(coords_ref, params_ref):
    x = coords_ref[...]
    mn = jnp.min(x, axis=2)
    mx = jnp.max(x, axis=2)
    rg = mx - mn + 1e-6
    pad = jnp.zeros((x.shape[0], 1), jnp.float32)
    rows8 = jnp.concatenate([mn, pad, rg, pad], axis=1)
    params_ref[...] = jnp.broadcast_to(rows8[:, :, None], params_ref.shape)


def _featsum_body(feat_ref, out_ref):
    out_ref[...] = jnp.sum(feat_ref[...], axis=1)[None]


def _softmax(x):
    m = jnp.max(x, axis=-1, keepdims=True)
    e = jnp.exp(x - m)
    return e / jnp.sum(e, axis=-1, keepdims=True)


def _final_body(hp_ref, sums_ref, w1_ref, b1_ref, w2_ref, b2_ref, gum_ref,
                lognf_ref, logits_ref, rwe_ref, rwt_ref, *, B, N, nparts):
    counts = hp_ref[...].reshape(B, nparts, _NBINS).sum(axis=1)
    hist = counts / (jnp.float32(N) + 1e-6)
    dmean = jnp.mean(hist, axis=1, keepdims=True)
    d = hist - dmean
    var = jnp.sum(d * d, axis=1, keepdims=True) / (_NBINS - 1)
    skew = jnp.mean(d * d * d, axis=1, keepdims=True) / (var * jnp.sqrt(var) + 1e-6)
    g_feat = jnp.sum(sums_ref[...], axis=0) / jnp.float32(N)
    ri = jnp.concatenate([g_feat, lognf_ref[...], dmean, var, skew], axis=1)
    dn = (((1,), (1,)), ((), ()))
    h = jnp.maximum(
        lax.dot_general(ri, w1_ref[...], dn, preferred_element_type=jnp.float32)
        + b1_ref[...], 0.0)
    logits = (lax.dot_general(h, w2_ref[...], dn, preferred_element_type=jnp.float32)
              + b2_ref[...])
    logits_ref[...] = logits
    rwe_ref[...] = _softmax(logits)
    rwt_ref[...] = _softmax(logits + gum_ref[...])


def _sc_hist_body(coords_hbm, params_hbm, out_hbm, cbuf, pbuf, hist,
                  *, npts, nper, N):
    nc = 2
    wid = lax.axis_index("s") * nc + lax.axis_index("c")
    b = wid // nper
    k = wid % nper
    for d in range(3):
        pltpu.sync_copy(
            coords_hbm.at[pl.ds((b * 3 + d) * N + k * npts, npts)],
            cbuf.at[pl.ds(d * npts, npts)])
    pltpu.sync_copy(params_hbm.at[pl.ds(b * 128, 128)], pbuf)

    zeros16 = jnp.zeros((16,), jnp.float32)

    def zero_body(j, carry):
        hist[pl.ds(j * 16, 16)] = zeros16
        return carry

    lax.fori_loop(0, _NBINS // 16, zero_body, 0)

    mnx = pbuf[pl.ds(0, 16)]
    mny = pbuf[pl.ds(16, 16)]
    mnz = pbuf[pl.ds(32, 16)]
    rgx = pbuf[pl.ds(64, 16)]
    rgy = pbuf[pl.ds(80, 16)]
    rgz = pbuf[pl.ds(96, 16)]
    ones16 = jnp.ones((16,), jnp.float32)
    g1 = jnp.float32(_GRID - 1)

    def body(i, carry):
        x = cbuf[pl.ds(i * 16, 16)]
        y = cbuf[pl.ds(npts + i * 16, 16)]
        z = cbuf[pl.ds(2 * npts + i * 16, 16)]
        ix = jnp.clip(((x - mnx) / rgx * g1).astype(jnp.int32), 0, _GRID - 1)
        iy = jnp.clip(((y - mny) / rgy * g1).astype(jnp.int32), 0, _GRID - 1)
        iz = jnp.clip(((z - mnz) / rgz * g1).astype(jnp.int32), 0, _GRID - 1)
        idx = ix * (_GRID * _GRID) + iy * _GRID + iz
        plsc.addupdate_scatter(hist, [idx], ones16)
        return carry

    lax.fori_loop(0, npts // 16, body, 0)
    pltpu.sync_copy(hist, out_hbm.at[pl.ds(wid * _NBINS, _NBINS)])


def _sc_hist(coords_flat, params_flat, B, N):
    npts = (B * N) // _NW
    nper = _NW // B
    mesh = plsc.VectorSubcoreMesh(core_axis_name="c", subcore_axis_name="s")
    body = functools.partial(_sc_hist_body, npts=npts, nper=nper, N=N)
    fn = pl.kernel(
        body,
        mesh=mesh,
        compiler_params=pltpu.CompilerParams(needs_layout_passes=False),
        out_type=jax.ShapeDtypeStruct((_NW * _NBINS,), jnp.float32),
        scratch_types=[
            pltpu.VMEM((3 * npts,), jnp.float32),
            pltpu.VMEM((128,), jnp.float32),
            pltpu.VMEM((_NBINS,), jnp.float32),
        ],
    )
    return fn(coords_flat, params_flat)


def kernel(feat_3d, coords, training, W1, b1, W2, b2):
    B, N, C = feat_3d.shape
    coords_t = jnp.transpose(coords, (0, 2, 1))

    params = pl.pallas_call(
        _minmax_body,
        out_shape=jax.ShapeDtypeStruct((B, 8, 16), jnp.float32),
    )(coords_t)

    hp = _sc_hist(coords_t.reshape(-1), params.reshape(-1), B, N)
    hp = hp.reshape(_NW, _NBINS)

    chunk = 4096
    nsteps = N // chunk
    sums_p = pl.pallas_call(
        _featsum_body,
        grid=(nsteps,),
        in_specs=[pl.BlockSpec((B, chunk, C), lambda i: (0, i, 0))],
        out_specs=pl.BlockSpec((1, B, C), lambda i: (i, 0, 0)),
        out_shape=jax.ShapeDtypeStruct((nsteps, B, C), jnp.float32),
        compiler_params=pltpu.CompilerParams(
            dimension_semantics=("parallel",)),
    )(feat_3d)

    log_n = (jnp.log(jnp.asarray(N, jnp.float32)) - 8.0) / 4.0
    lognf = jnp.broadcast_to(log_n.reshape(1, 1), (B, 1))
    u = jax.random.uniform(jax.random.key(42), (B, 3), dtype=jnp.float32)
    gumbel = -jnp.log(-jnp.log(u + 1e-10) + 1e-10)

    out_sd = jax.ShapeDtypeStruct((B, 3), jnp.float32)
    final = functools.partial(_final_body, B=B, N=N, nparts=_NW // B)
    logits, rwe, rwt = pl.pallas_call(
        final,
        out_shape=[out_sd, out_sd, out_sd],
    )(hp, sums_p, W1, b1.reshape(1, -1), W2, b2.reshape(1, -1), gumbel, lognf)

    routing_weights = jnp.where(training, rwt, rwe)
    return routing_weights, logits
